# Initial kernel scaffold; baseline (speedup 1.0000x reference)
import jax
import jax.numpy as jnp
from jax.experimental import pallas as pl


def kernel(inputs, targets, all_posvid):
    raise NotImplementedError("write your pallas kernel here")


# speedup vs baseline: 2.1452x; 2.1452x over previous
import functools

import jax
import jax.numpy as jnp
from jax.experimental import pallas as pl

_EPS = 0.1


def _row_stats_body(x_ref, t_ref, loss_ref):
    x = x_ref[...]
    m = jnp.max(x, axis=1, keepdims=True)
    s = jnp.sum(jnp.exp(x - m), axis=1, keepdims=True)
    lse = m + jnp.log(s)
    rowsum = jnp.sum(x, axis=1, keepdims=True)
    ids = jax.lax.broadcasted_iota(jnp.int32, x.shape, 1)
    tgt = t_ref[...]
    tval = jnp.sum(jnp.where(ids == tgt, x, 0.0), axis=1, keepdims=True)
    C = x.shape[1]
    loss_ref[...] = lse - (1.0 - _EPS) * tval - (_EPS / C) * rowsum


@jax.jit
def kernel(inputs, targets, all_posvid):
    del all_posvid
    B, C = inputs.shape
    RB = 8
    loss_rows = pl.pallas_call(
        _row_stats_body,
        grid=(B // RB,),
        in_specs=[
            pl.BlockSpec((RB, C), lambda i: (i, 0)),
            pl.BlockSpec((RB, 1), lambda i: (i, 0)),
        ],
        out_specs=pl.BlockSpec((RB, 1), lambda i: (i, 0)),
        out_shape=jax.ShapeDtypeStruct((B, 1), jnp.float32),
    )(inputs, targets.reshape(B, 1))
    return jnp.mean(loss_rows)
